# baseline (speedup 1.0000x reference)
import jax
import jax.numpy as jnp
from jax import lax
from jax.experimental import pallas as pl
from jax.experimental.pallas import tpu as pltpu
from jax.experimental.pallas import tpu_sc as plsc

NC = 2
NS = 16
NW = NC * NS

EMBED_DIM = 64
CHAR_EMBED_DIM = 16
CHAR_SIZE = 262
BATCH = 1024
SEQ = 200
WORD_LEN = 16

BB = BATCH // 128
UNITS = SEQ * BB
U_PER = UNITS // NW
PAIRS = U_PER // 2


def _body(widx_hbm, chars_hbm, wtab_hbm, ctab_hbm, wout_hbm, cout_hbm,
          wi0, wi1, wg0, wg1, wrows0, wrows1, wo0, wo1,
          cidx0, cidx1, ctab_v, co0, co1,
          wisem, cisem, wgsem, wosem0, wosem1, cosem0, cosem1):
    wid = lax.axis_index("s") * NC + lax.axis_index("c")
    ubase = wid * U_PER

    iota = lax.iota(jnp.int32, 16)
    rows_g = [iota + g * 16 for g in range(8)]

    wi = (wi0, wi1)
    wg = (wg0, wg1)
    wrows = (wrows0, wrows1)
    wo = (wo0, wo1)
    wosem = (wosem0, wosem1)
    cidx = (cidx0, cidx1)
    co = (co0, co1)
    cosem = (cosem0, cosem1)

    def su(t):
        u = ubase + t
        return u // BB, u % BB

    def widx_issue(t, j):
        pltpu.async_copy(widx_hbm.at[ubase + t], wi[j], wisem)

    def widx_wait(j):
        pltpu.make_async_copy(widx_hbm.at[0], wi[j], wisem).wait()

    def cidx_issue(t, j):
        s, tb = su(t)
        pltpu.async_copy(chars_hbm.at[s, :, pl.ds(tb * 128, 128)], cidx[j],
                         cisem)

    def cidx_wait(j):
        pltpu.make_async_copy(chars_hbm.at[0, :, pl.ds(0, 128)], cidx[j],
                              cisem).wait()

    def wgather_issue(j):
        pltpu.async_copy(wtab_hbm.at[wg[j]], wrows[j], wgsem)

    def wgather_wait(j):
        pltpu.make_async_copy(wtab_hbm.at[pl.ds(0, 128)], wrows[j],
                              wgsem).wait()

    def wout_issue(t, j):
        s, bb = su(t)
        pltpu.async_copy(wo[j], wout_hbm.at[s, :, pl.ds(bb, 1)], wosem[j])

    def wout_wait(j):
        pltpu.make_async_copy(wo[j], wout_hbm.at[0, :, pl.ds(0, 1)],
                              wosem[j]).wait()

    def cout_issue(t, j):
        s, tb = su(t)
        pltpu.async_copy(co[j], cout_hbm.at[s, :, :, pl.ds(tb, 1)], cosem[j])

    def cout_wait(j):
        pltpu.make_async_copy(co[j], cout_hbm.at[0, :, :, pl.ds(0, 1)],
                              cosem[j]).wait()

    def wprep(j):
        for g in range(8):
            ids = wi[j][pl.ds(g * 16, 16)]
            wg[j][pl.ds(g * 16, 16)] = lax.shift_right_logical(ids, 1)

    def wcompute(j):
        rbuf = wrows[j]
        obuf = wo[j]
        par64 = [
            lax.shift_left(wi[j][pl.ds(g * 16, 16)] & 1, 6) for g in range(8)
        ]

        @pl.loop(0, 8)
        def _cb(cb):
            for ci in range(8):
                c = cb * 8 + ci
                for g in range(8):
                    vals = plsc.load_gather(rbuf, [rows_g[g], par64[g] + c])
                    obuf[cb, 0, ci, pl.ds(g * 16, 16)] = vals

    def ccompute(j):
        ibuf = cidx[j]
        obuf = co[j]

        @pl.loop(0, WORD_LEN)
        def _w(w):
            idxv = [ibuf[w, pl.ds(g * 16, 16)] for g in range(8)]
            for tc in range(2):
                for ci in range(8):
                    c = tc * 8 + ci
                    cols = jnp.full((16,), c, jnp.int32)
                    for g in range(8):
                        vals = plsc.load_gather(ctab_v, [cols, idxv[g]])
                        obuf[w, tc, 0, ci, pl.ds(g * 16, 16)] = vals

    pltpu.sync_copy(ctab_hbm, ctab_v)
    s0, tb0 = ubase // BB, ubase % BB
    pltpu.sync_copy(widx_hbm.at[ubase], wi0)
    pltpu.sync_copy(chars_hbm.at[s0, :, pl.ds(tb0 * 128, 128)], cidx0)

    @pl.loop(0, PAIRS)
    def _pair(p):
        for j in range(2):
            t = p * 2 + j

            wprep(j)
            wgather_issue(j)

            if j == 0:
                widx_issue(t + 1, 1)
                cidx_issue(t + 1, 1)
            else:
                @pl.when(p < PAIRS - 1)
                def _():
                    widx_issue(t + 1, 0)
                    cidx_issue(t + 1, 0)

            @pl.when(p > 0)
            def _():
                cout_wait(j)

            ccompute(j)
            cout_issue(t, j)

            @pl.when(p > 0)
            def _():
                wout_wait(j)

            wgather_wait(j)
            wcompute(j)
            wout_issue(t, j)

            if j == 0:
                widx_wait(1)
                cidx_wait(1)
            else:
                @pl.when(p < PAIRS - 1)
                def _():
                    widx_wait(0)
                    cidx_wait(0)

    cout_wait(0)
    cout_wait(1)
    wout_wait(0)
    wout_wait(1)


@jax.jit
def _run(words, chars, word_table, char_table):
    widx2 = words.T.reshape(UNITS, 128)
    charsT = chars.transpose(1, 2, 0)
    ctabT = char_table.T
    wtab2 = word_table.reshape(500000, 128)

    f = pl.kernel(
        _body,
        out_type=(
            jax.ShapeDtypeStruct((SEQ, 8, BB, 8, 128), jnp.float32),
            jax.ShapeDtypeStruct((SEQ, WORD_LEN, 2, BB, 8, 128), jnp.float32),
        ),
        mesh=plsc.VectorSubcoreMesh(core_axis_name="c", subcore_axis_name="s"),
        compiler_params=pltpu.CompilerParams(use_tc_tiling_on_sc=False,
                                             needs_layout_passes=False),
        scratch_types=[
            pltpu.VMEM((128,), jnp.int32),
            pltpu.VMEM((128,), jnp.int32),
            pltpu.VMEM((128,), jnp.int32),
            pltpu.VMEM((128,), jnp.int32),
            pltpu.VMEM((128, 128), jnp.float32),
            pltpu.VMEM((128, 128), jnp.float32),
            pltpu.VMEM((8, 1, 8, 128), jnp.float32),
            pltpu.VMEM((8, 1, 8, 128), jnp.float32),
            pltpu.VMEM((WORD_LEN, 128), jnp.int32),
            pltpu.VMEM((WORD_LEN, 128), jnp.int32),
            pltpu.VMEM((CHAR_EMBED_DIM, CHAR_SIZE), jnp.float32),
            pltpu.VMEM((WORD_LEN, 2, 1, 8, 128), jnp.float32),
            pltpu.VMEM((WORD_LEN, 2, 1, 8, 128), jnp.float32),
        ] + [pltpu.SemaphoreType.DMA] * 7,
    )
    kw, kc = f(widx2, charsT, wtab2, ctabT)

    word_emb = kw.transpose(2, 4, 0, 1, 3).reshape(BATCH, SEQ, EMBED_DIM)
    char_emb = kc.transpose(3, 5, 0, 1, 2, 4).reshape(
        BATCH, SEQ, WORD_LEN, CHAR_EMBED_DIM)
    return word_emb, char_emb


def kernel(words, chars, word_table, char_table):
    return _run(words, chars, word_table, char_table)


# speedup vs baseline: 1.3182x; 1.3182x over previous
import jax
import jax.numpy as jnp
from jax import lax
from jax.experimental import pallas as pl
from jax.experimental.pallas import tpu as pltpu
from jax.experimental.pallas import tpu_sc as plsc

NC = 2
NS = 16
NW = NC * NS

EMBED_DIM = 64
CHAR_EMBED_DIM = 16
CHAR_SIZE = 262
BATCH = 1024
SEQ = 200
WORD_LEN = 16

BB = BATCH // 128
UNITS = SEQ * BB
U_PER = UNITS // NW
PAIRS = U_PER // 2

_MESH = dict(
    mesh=plsc.VectorSubcoreMesh(core_axis_name="c", subcore_axis_name="s"),
    compiler_params=pltpu.CompilerParams(use_tc_tiling_on_sc=False,
                                         needs_layout_passes=False),
)


def _char_body(chars_hbm, ctab_hbm, cout_hbm,
               cidx0, cidx1, ctab_v, co0, co1,
               cisem, cosem0, cosem1):
    wid = lax.axis_index("s") * NC + lax.axis_index("c")
    ubase = wid * U_PER

    cidx = (cidx0, cidx1)
    co = (co0, co1)
    cosem = (cosem0, cosem1)

    def su(t):
        u = ubase + t
        return u // BB, u % BB

    def cidx_issue(t, j):
        s, tb = su(t)
        pltpu.async_copy(chars_hbm.at[s, :, pl.ds(tb * 128, 128)], cidx[j],
                         cisem)

    def cidx_wait(j):
        pltpu.make_async_copy(chars_hbm.at[0, :, pl.ds(0, 128)], cidx[j],
                              cisem).wait()

    def cout_issue(t, j):
        s, tb = su(t)
        pltpu.async_copy(co[j], cout_hbm.at[s, :, :, pl.ds(tb, 1)], cosem[j])

    def cout_wait(j):
        pltpu.make_async_copy(co[j], cout_hbm.at[0, :, :, pl.ds(0, 1)],
                              cosem[j]).wait()

    def ccompute(j):
        ibuf = cidx[j]
        obuf = co[j]

        @pl.loop(0, WORD_LEN)
        def _w(w):
            idxv = [ibuf[w, pl.ds(g * 16, 16)] for g in range(8)]
            for tc in range(2):
                for ci in range(8):
                    c = tc * 8 + ci
                    cols = jnp.full((16,), c, jnp.int32)
                    for g in range(8):
                        vals = plsc.load_gather(ctab_v, [cols, idxv[g]])
                        obuf[w, tc, 0, ci, pl.ds(g * 16, 16)] = vals

    pltpu.sync_copy(ctab_hbm, ctab_v)
    s0, tb0 = ubase // BB, ubase % BB
    pltpu.sync_copy(chars_hbm.at[s0, :, pl.ds(tb0 * 128, 128)], cidx0)

    @pl.loop(0, PAIRS)
    def _pair(p):
        for j in range(2):
            t = p * 2 + j

            if j == 0:
                cidx_issue(t + 1, 1)
            else:
                @pl.when(p < PAIRS - 1)
                def _():
                    cidx_issue(t + 1, 0)

            @pl.when(p > 0)
            def _():
                cout_wait(j)

            ccompute(j)
            cout_issue(t, j)

            if j == 0:
                cidx_wait(1)
            else:
                @pl.when(p < PAIRS - 1)
                def _():
                    cidx_wait(0)

    cout_wait(0)
    cout_wait(1)


def _word_body(widx_hbm, wtab_hbm, wout_hbm,
               wi0, wi1, wg0, wg1, wrows0, wrows1, wo0, wo1,
               wisem, wgsem, wosem0, wosem1):
    wid = lax.axis_index("s") * NC + lax.axis_index("c")
    ubase = wid * U_PER

    iota = lax.iota(jnp.int32, 16)
    rows_g = [iota + g * 16 for g in range(8)]

    wi = (wi0, wi1)
    wg = (wg0, wg1)
    wrows = (wrows0, wrows1)
    wo = (wo0, wo1)
    wosem = (wosem0, wosem1)

    def su(t):
        u = ubase + t
        return u // BB, u % BB

    def widx_issue(t, j):
        pltpu.async_copy(widx_hbm.at[ubase + t], wi[j], wisem)

    def widx_wait(j):
        pltpu.make_async_copy(widx_hbm.at[0], wi[j], wisem).wait()

    def wgather_issue(j):
        pltpu.async_copy(wtab_hbm.at[wg[j]], wrows[j], wgsem)

    def wgather_wait(j):
        pltpu.make_async_copy(wtab_hbm.at[pl.ds(0, 128)], wrows[j],
                              wgsem).wait()

    def wout_issue(t, j):
        s, bb = su(t)
        pltpu.async_copy(wo[j], wout_hbm.at[s, :, pl.ds(bb, 1)], wosem[j])

    def wout_wait(j):
        pltpu.make_async_copy(wo[j], wout_hbm.at[0, :, pl.ds(0, 1)],
                              wosem[j]).wait()

    def wprep(j):
        for g in range(8):
            ids = wi[j][pl.ds(g * 16, 16)]
            wg[j][pl.ds(g * 16, 16)] = lax.shift_right_logical(ids, 1)

    def wcompute(j):
        rbuf = wrows[j]
        obuf = wo[j]
        par64 = [
            lax.shift_left(wi[j][pl.ds(g * 16, 16)] & 1, 6) for g in range(8)
        ]

        @pl.loop(0, 8)
        def _cb(cb):
            for ci in range(8):
                c = cb * 8 + ci
                for g in range(8):
                    vals = plsc.load_gather(rbuf, [rows_g[g], par64[g] + c])
                    obuf[cb, 0, ci, pl.ds(g * 16, 16)] = vals

    pltpu.sync_copy(widx_hbm.at[ubase], wi0)
    wprep(0)
    wgather_issue(0)
    widx_issue(1, 1)

    @pl.loop(0, PAIRS)
    def _pair(p):
        for j in range(2):
            t = p * 2 + j

            if j == 0:
                widx_wait(1)
                wprep(1)
                wgather_issue(1)
            else:
                @pl.when(p < PAIRS - 1)
                def _():
                    widx_wait(0)
                    wprep(0)
                    wgather_issue(0)

            @pl.when(p > 0)
            def _():
                wout_wait(j)

            wgather_wait(j)
            wcompute(j)
            wout_issue(t, j)

            @pl.when(p < PAIRS - 1)
            def _():
                widx_issue(t + 2, j)

    wout_wait(0)
    wout_wait(1)


@jax.jit
def _run(words, chars, word_table, char_table):
    widx2 = words.T.reshape(UNITS, 128)
    charsT = chars.transpose(1, 2, 0)
    ctabT = char_table.T
    wtab2 = word_table.reshape(500000, 128)

    fc = pl.kernel(
        _char_body,
        out_type=jax.ShapeDtypeStruct((SEQ, WORD_LEN, 2, BB, 8, 128),
                                      jnp.float32),
        scratch_types=[
            pltpu.VMEM((WORD_LEN, 128), jnp.int32),
            pltpu.VMEM((WORD_LEN, 128), jnp.int32),
            pltpu.VMEM((CHAR_EMBED_DIM, CHAR_SIZE), jnp.float32),
            pltpu.VMEM((WORD_LEN, 2, 1, 8, 128), jnp.float32),
            pltpu.VMEM((WORD_LEN, 2, 1, 8, 128), jnp.float32),
        ] + [pltpu.SemaphoreType.DMA] * 3,
        **_MESH,
    )
    fw = pl.kernel(
        _word_body,
        out_type=jax.ShapeDtypeStruct((SEQ, 8, BB, 8, 128), jnp.float32),
        scratch_types=[
            pltpu.VMEM((128,), jnp.int32),
            pltpu.VMEM((128,), jnp.int32),
            pltpu.VMEM((128,), jnp.int32),
            pltpu.VMEM((128,), jnp.int32),
            pltpu.VMEM((128, 128), jnp.float32),
            pltpu.VMEM((128, 128), jnp.float32),
            pltpu.VMEM((8, 1, 8, 128), jnp.float32),
            pltpu.VMEM((8, 1, 8, 128), jnp.float32),
        ] + [pltpu.SemaphoreType.DMA] * 4,
        **_MESH,
    )

    kc = fc(charsT, ctabT)
    kw = fw(widx2, wtab2)

    word_emb = kw.transpose(2, 4, 0, 1, 3).reshape(BATCH, SEQ, EMBED_DIM)
    char_emb = kc.transpose(3, 5, 0, 1, 2, 4).reshape(
        BATCH, SEQ, WORD_LEN, CHAR_EMBED_DIM)
    return word_emb, char_emb


def kernel(words, chars, word_table, char_table):
    return _run(words, chars, word_table, char_table)


# speedup vs baseline: 1.3415x; 1.0177x over previous
import jax
import jax.numpy as jnp
from jax import lax
from jax.experimental import pallas as pl
from jax.experimental.pallas import tpu as pltpu
from jax.experimental.pallas import tpu_sc as plsc

NC = 2
NS = 16
NW = NC * NS

EMBED_DIM = 64
CHAR_EMBED_DIM = 16
CHAR_SIZE = 262
BATCH = 1024
SEQ = 200
WORD_LEN = 16

BB = BATCH // 128
UNITS = SEQ * BB
U_PER = UNITS // NW
PAIRS = U_PER // 2

_MESH = dict(
    mesh=plsc.VectorSubcoreMesh(core_axis_name="c", subcore_axis_name="s"),
    compiler_params=pltpu.CompilerParams(use_tc_tiling_on_sc=False,
                                         needs_layout_passes=False),
)


def _char_body(chars_hbm, ctab_hbm, cout_hbm,
               cidx0, cidx1, ctab_v, co0, co1,
               cisem, cosem0, cosem1):
    wid = lax.axis_index("s") * NC + lax.axis_index("c")
    ubase = wid * U_PER

    cidx = (cidx0, cidx1)
    co = (co0, co1)
    cosem = (cosem0, cosem1)

    def su(t):
        u = ubase + t
        return u // BB, u % BB

    def cidx_issue(t, j):
        s, tb = su(t)
        pltpu.async_copy(chars_hbm.at[s, :, pl.ds(tb * 128, 128)], cidx[j],
                         cisem)

    def cidx_wait(j):
        pltpu.make_async_copy(chars_hbm.at[0, :, pl.ds(0, 128)], cidx[j],
                              cisem).wait()

    def cout_issue(t, j):
        s, tb = su(t)
        pltpu.async_copy(co[j], cout_hbm.at[s, :, :, pl.ds(tb, 1)], cosem[j])

    def cout_wait(j):
        pltpu.make_async_copy(co[j], cout_hbm.at[0, :, :, pl.ds(0, 1)],
                              cosem[j]).wait()

    def ccompute(j):
        ibuf = cidx[j]
        obuf = co[j]

        @pl.loop(0, WORD_LEN)
        def _w(w):
            idxv = [ibuf[w, pl.ds(g * 16, 16)] for g in range(8)]
            for tc in range(2):
                for ci in range(8):
                    c = tc * 8 + ci
                    cols = jnp.full((16,), c, jnp.int32)
                    for g in range(8):
                        vals = plsc.load_gather(ctab_v, [cols, idxv[g]])
                        obuf[w, tc, 0, ci, pl.ds(g * 16, 16)] = vals

    pltpu.sync_copy(ctab_hbm, ctab_v)
    s0, tb0 = ubase // BB, ubase % BB
    pltpu.sync_copy(chars_hbm.at[s0, :, pl.ds(tb0 * 128, 128)], cidx0)

    @pl.loop(0, PAIRS)
    def _pair(p):
        for j in range(2):
            t = p * 2 + j

            if j == 0:
                cidx_issue(t + 1, 1)
            else:
                @pl.when(p < PAIRS - 1)
                def _():
                    cidx_issue(t + 1, 0)

            @pl.when(p > 0)
            def _():
                cout_wait(j)

            ccompute(j)
            cout_issue(t, j)

            if j == 0:
                cidx_wait(1)
            else:
                @pl.when(p < PAIRS - 1)
                def _():
                    cidx_wait(0)

    cout_wait(0)
    cout_wait(1)


NBUF = 5
WBLK = U_PER // NBUF


def _word_body(widx_hbm, wtab_hbm, wout_hbm, widx_v, *bufs):
    wrows = bufs[0:NBUF]
    wo = bufs[NBUF:2 * NBUF]
    gsem = bufs[2 * NBUF:3 * NBUF]
    osem = bufs[3 * NBUF:4 * NBUF]

    wid = lax.axis_index("s") * NC + lax.axis_index("c")
    ubase = wid * U_PER

    iota = lax.iota(jnp.int32, 16)
    rows_g = [iota + g * 16 for g in range(8)]

    def wgather_issue(t, j):
        pltpu.async_copy(wtab_hbm.at[widx_v.at[t]], wrows[j], gsem[j])

    def wgather_wait(j):
        pltpu.make_async_copy(wtab_hbm.at[pl.ds(0, 128)], wrows[j],
                              gsem[j]).wait()

    def wout_issue(t, j):
        u = ubase + t
        pltpu.async_copy(wo[j], wout_hbm.at[u // BB, :, pl.ds(u % BB, 1)],
                         osem[j])

    def wout_wait(j):
        pltpu.make_async_copy(wo[j], wout_hbm.at[0, :, pl.ds(0, 1)],
                              osem[j]).wait()

    def wcompute(j):
        rbuf = wrows[j]
        obuf = wo[j]

        @pl.loop(0, 8)
        def _cb(cb):
            for ci in range(8):
                c = cb * 8 + ci
                cols = jnp.full((16,), c, jnp.int32)
                for g in range(8):
                    vals = plsc.load_gather(rbuf, [rows_g[g], cols])
                    obuf[cb, 0, ci, pl.ds(g * 16, 16)] = vals

    pltpu.sync_copy(widx_hbm.at[pl.ds(ubase, U_PER)], widx_v)
    for k in range(NBUF):
        wgather_issue(k, k)

    @pl.loop(0, WBLK)
    def _blk(p):
        for j in range(NBUF):
            t = p * NBUF + j
            wgather_wait(j)

            @pl.when(p > 0)
            def _():
                wout_wait(j)

            wcompute(j)
            wout_issue(t, j)

            @pl.when(p < WBLK - 1)
            def _():
                wgather_issue(t + NBUF, j)

    for k in range(NBUF):
        wout_wait(k)


@jax.jit
def _run(words, chars, word_table, char_table):
    widx2 = words.T.reshape(UNITS, 128)
    charsT = chars.transpose(1, 2, 0)
    ctabT = char_table.T

    fc = pl.kernel(
        _char_body,
        out_type=jax.ShapeDtypeStruct((SEQ, WORD_LEN, 2, BB, 8, 128),
                                      jnp.float32),
        scratch_types=[
            pltpu.VMEM((WORD_LEN, 128), jnp.int32),
            pltpu.VMEM((WORD_LEN, 128), jnp.int32),
            pltpu.VMEM((CHAR_EMBED_DIM, CHAR_SIZE), jnp.float32),
            pltpu.VMEM((WORD_LEN, 2, 1, 8, 128), jnp.float32),
            pltpu.VMEM((WORD_LEN, 2, 1, 8, 128), jnp.float32),
        ] + [pltpu.SemaphoreType.DMA] * 3,
        **_MESH,
    )
    fw = pl.kernel(
        _word_body,
        out_type=jax.ShapeDtypeStruct((SEQ, 8, BB, 8, 128), jnp.float32),
        scratch_types=(
            [pltpu.VMEM((U_PER, 128), jnp.int32)]
            + [pltpu.VMEM((128, EMBED_DIM), jnp.float32)] * NBUF
            + [pltpu.VMEM((8, 1, 8, 128), jnp.float32)] * NBUF
            + [pltpu.SemaphoreType.DMA] * (2 * NBUF)
        ),
        **_MESH,
    )

    kc = fc(charsT, ctabT)
    kw = fw(widx2, word_table)

    word_emb = kw.transpose(2, 4, 0, 1, 3).reshape(BATCH, SEQ, EMBED_DIM)
    char_emb = kc.transpose(3, 5, 0, 1, 2, 4).reshape(
        BATCH, SEQ, WORD_LEN, CHAR_EMBED_DIM)
    return word_emb, char_emb


def kernel(words, chars, word_table, char_table):
    return _run(words, chars, word_table, char_table)
